# baseline (speedup 1.0000x reference)
import jax
import jax.numpy as jnp
from jax.experimental import pallas as pl
from jax.experimental.pallas import tpu as pltpu

_BLOCK = 2000


def _dot_t(lhs, rhs):
    return jax.lax.dot_general(lhs, rhs, (((0,), (1,)), ((), ())),
                               preferred_element_type=jnp.float32)


def _dot_rt(lhs, rhs):
    return jax.lax.dot_general(lhs, rhs, (((1,), (1,)), ((), ())),
                               preferred_element_type=jnp.float32)


def _attr_block(val_ref, boo_ref, tw_ref, de_ref,
                w1_ref, w2_ref, w3_ref, w4_ref, win_ref, wo_ref, wr_ref,
                bx_ref, bin_ref, bo_ref, br_ref,
                o_ref, a1_s, a2_s, a3_s, a4_s, c_s):
    fd = wr_ref.shape[0]
    e = wo_ref.shape[0]

    @pl.when(pl.program_id(0) == 0)
    def _fold():
        wv = win_ref[2 * e:3 * e, :]
        wr_wo = jnp.dot(wr_ref[...], wo_ref[...],
                        preferred_element_type=jnp.float32)
        m_t = jnp.dot(wr_wo, wv, preferred_element_type=jnp.float32)
        a1_s[...] = _dot_t(w1_ref[...], m_t[:, 0 * fd:1 * fd])
        a2_s[...] = _dot_t(w2_ref[...], m_t[:, 1 * fd:2 * fd])
        a3_s[...] = _dot_t(w3_ref[...], m_t[:, 2 * fd:3 * fd])
        a4_s[...] = _dot_t(w4_ref[...], m_t[:, 3 * fd:4 * fd])
        bv = bin_ref[:, 2 * e:3 * e]
        c_s[...] = (_dot_rt(bx_ref[...], m_t) + _dot_rt(bv, wr_wo)
                    + _dot_rt(bo_ref[...], wr_ref[...]) + br_ref[...])

    acc = jnp.dot(tw_ref[...], a3_s[...], preferred_element_type=jnp.float32)
    acc = acc + jnp.dot(de_ref[...], a4_s[...], preferred_element_type=jnp.float32)
    acc = acc + jnp.dot(val_ref[...], a1_s[...], preferred_element_type=jnp.float32)
    acc = acc + jnp.dot(boo_ref[...], a2_s[...], preferred_element_type=jnp.float32)
    acc = acc + c_s[...]
    o_ref[...] = jnp.where(acc >= 0.0, acc, 0.01 * acc)


def kernel(value_feats, bool_feats, tweet_feats, des_feats,
           W1, b1, W2, b2, W3, b3, W4, b4,
           W_in, b_in, W_out, b_out, W_r, b_r):
    N, VN = value_feats.shape
    BN = bool_feats.shape[1]
    TN = tweet_feats.shape[1]
    DN = des_feats.shape[1]
    FD = W_r.shape[0]
    E = W_out.shape[0]

    bx = jnp.concatenate([b1, b2, b3, b4]).reshape(1, E)
    bin2 = b_in.reshape(1, 3 * E)
    bo2 = b_out.reshape(1, E)
    br2 = b_r.reshape(1, FD)

    full = lambda shape: pl.BlockSpec(shape, lambda i: (0, 0))
    grid = (pl.cdiv(N, _BLOCK),)
    out = pl.pallas_call(
        _attr_block,
        grid=grid,
        in_specs=[
            pl.BlockSpec((_BLOCK, VN), lambda i: (i, 0)),
            pl.BlockSpec((_BLOCK, BN), lambda i: (i, 0)),
            pl.BlockSpec((_BLOCK, TN), lambda i: (i, 0)),
            pl.BlockSpec((_BLOCK, DN), lambda i: (i, 0)),
            full((FD, VN)), full((FD, BN)), full((FD, TN)), full((FD, DN)),
            full((3 * E, E)), full((E, E)), full((FD, E)),
            full((1, E)), full((1, 3 * E)), full((1, E)), full((1, FD)),
        ],
        out_specs=pl.BlockSpec((_BLOCK, FD), lambda i: (i, 0)),
        out_shape=jax.ShapeDtypeStruct((N, FD), jnp.float32),
        scratch_shapes=[
            pltpu.VMEM((VN, FD), jnp.float32),
            pltpu.VMEM((BN, FD), jnp.float32),
            pltpu.VMEM((TN, FD), jnp.float32),
            pltpu.VMEM((DN, FD), jnp.float32),
            pltpu.VMEM((1, FD), jnp.float32),
        ],
    )(value_feats, bool_feats, tweet_feats, des_feats,
      W1, W2, W3, W4, W_in, W_out, W_r, bx, bin2, bo2, br2)
    return out


# speedup vs baseline: 1.3268x; 1.3268x over previous
import jax
import jax.numpy as jnp
from jax.experimental import pallas as pl

_BLOCK = 2048


def _attr_block(vbt_ref, tw_ref, de_ref, a12_ref, a3_ref, a4_ref, c_ref, o_ref):
    acc = jnp.dot(tw_ref[...], a3_ref[...], preferred_element_type=jnp.float32)
    acc = acc + jnp.dot(de_ref[...], a4_ref[...], preferred_element_type=jnp.float32)
    acc = acc + jax.lax.dot_general(vbt_ref[...], a12_ref[...],
                                    (((0,), (0,)), ((), ())),
                                    preferred_element_type=jnp.float32)
    acc = acc + c_ref[...]
    o_ref[...] = jnp.where(acc >= 0.0, acc, 0.01 * acc)


def kernel(value_feats, bool_feats, tweet_feats, des_feats,
           W1, b1, W2, b2, W3, b3, W4, b4,
           W_in, b_in, W_out, b_out, W_r, b_r):
    N, VN = value_feats.shape
    BN = bool_feats.shape[1]
    TN = tweet_feats.shape[1]
    DN = des_feats.shape[1]
    FD = W_r.shape[0]
    E = W_out.shape[0]

    Wv = W_in[2 * E:3 * E]
    bv = b_in[2 * E:3 * E]
    m_t = W_r @ W_out @ Wv
    a1 = (m_t[:, 0 * FD:1 * FD] @ W1).T
    a2 = (m_t[:, 1 * FD:2 * FD] @ W2).T
    a3 = (m_t[:, 2 * FD:3 * FD] @ W3).T
    a4 = (m_t[:, 3 * FD:4 * FD] @ W4).T
    a12 = jnp.concatenate([a1, a2], axis=0)
    bx = jnp.concatenate([b1, b2, b3, b4])
    c = bx @ m_t.T + bv @ (W_r @ W_out).T + b_out @ W_r.T + b_r
    c2 = c.reshape(1, FD)
    vbt = jnp.concatenate([value_feats, bool_feats], axis=1).T

    grid = (pl.cdiv(N, _BLOCK),)
    out = pl.pallas_call(
        _attr_block,
        grid=grid,
        in_specs=[
            pl.BlockSpec((VN + BN, _BLOCK), lambda i: (0, i)),
            pl.BlockSpec((_BLOCK, TN), lambda i: (i, 0)),
            pl.BlockSpec((_BLOCK, DN), lambda i: (i, 0)),
            pl.BlockSpec((VN + BN, FD), lambda i: (0, 0)),
            pl.BlockSpec((TN, FD), lambda i: (0, 0)),
            pl.BlockSpec((DN, FD), lambda i: (0, 0)),
            pl.BlockSpec((1, FD), lambda i: (0, 0)),
        ],
        out_specs=pl.BlockSpec((_BLOCK, FD), lambda i: (i, 0)),
        out_shape=jax.ShapeDtypeStruct((N, FD), jnp.float32),
    )(vbt, tweet_feats, des_feats, a12, a3, a4, c2)
    return out
